# baseline (speedup 1.0000x reference)
import functools

import jax
import jax.numpy as jnp
from jax.experimental import pallas as pl
from jax.experimental.pallas import tpu as pltpu

IOU_THRESHOLD = 0.5
NEG_POS_RATIO = 3
LANE = 128


def _match_kernel(a_ref, p_ref, c_ref, gt_ref, out_ref, iou_sc, bm_sc,
                  neg_sc, stats_sc, *, n_valid, n_gt, chunk, batch):
    MB, S, L = a_ref.shape[0], a_ref.shape[2], a_ref.shape[3]
    f32 = jnp.float32
    padded = S * L > n_valid
    CH = chunk
    n_ch = S // CH

    lane = jax.lax.broadcasted_iota(jnp.int32, (1, 1, LANE), 2)

    def sl1(d):
        d = jnp.abs(d)
        return jnp.where(d < 1.0, 0.5 * d * d, d - 0.5)

    for b in range(MB):
        gb = pl.program_id(0) * MB + b
        gtv = gt_ref[b]
        gcoord = [[gtv[g:g + 1, c:c + 1] for c in range(4)]
                  for g in range(n_gt)]
        gareas = [(gc[2] - gc[0]) * (gc[3] - gc[1]) for gc in gcoord]
        for ci in range(n_ch):
            cs = ci * CH
            sl = slice(cs, cs + CH)
            ax0 = a_ref[b, 0, sl]
            ay0 = a_ref[b, 1, sl]
            ax1 = a_ref[b, 2, sl]
            ay1 = a_ref[b, 3, sl]
            area_a = (ax1 - ax0) * (ay1 - ay0)

            best_iou = jnp.full((CH, L), -1.0, f32)
            m_x0 = jnp.zeros((CH, L), f32)
            m_y0 = jnp.zeros((CH, L), f32)
            m_x1 = jnp.zeros((CH, L), f32)
            m_y1 = jnp.zeros((CH, L), f32)

            for g in range(n_gt):
                gx0, gy0, gx1, gy1 = gcoord[g]
                w = jnp.maximum(
                    jnp.minimum(ax1, gx1) - jnp.maximum(ax0, gx0), 0.0)
                h = jnp.maximum(
                    jnp.minimum(ay1, gy1) - jnp.maximum(ay0, gy0), 0.0)
                inter = w * h
                iou = inter / (area_a + gareas[g] - inter + 1e-9)
                iou_sc[g, sl] = iou
                upd = iou > best_iou
                best_iou = jnp.where(upd, iou, best_iou)
                m_x0 = jnp.where(upd, gx0, m_x0)
                m_y0 = jnp.where(upd, gy0, m_y0)
                m_x1 = jnp.where(upd, gx1, m_x1)
                m_y1 = jnp.where(upd, gy1, m_y1)

            bm_sc[0, sl] = best_iou
            bm_sc[1, sl] = m_x0
            bm_sc[2, sl] = m_y0
            bm_sc[3, sl] = m_x1
            bm_sc[4, sl] = m_y1

        row = jax.lax.broadcasted_iota(jnp.int32, (S, L), 0)
        col = jax.lax.broadcasted_iota(jnp.int32, (S, L), 1)
        fi3 = (row * L + col)[None]
        iou_all = iou_sc[...]
        gmax = jnp.max(iou_all, axis=(1, 2), keepdims=True)
        cand = jnp.where(iou_all == gmax, fi3, jnp.int32(2 ** 30))
        bi = jnp.min(cand, axis=(1, 2), keepdims=True)

        loc_sum = jnp.float32(0.0)
        n_pos = jnp.float32(0.0)
        pos_loss = jnp.float32(0.0)
        for ci in range(n_ch):
            cs = ci * CH
            sl = slice(cs, cs + CH)
            rowc = jax.lax.broadcasted_iota(jnp.int32, (CH, L), 0) + cs
            colc = jax.lax.broadcasted_iota(jnp.int32, (CH, L), 1)
            flat_c = (rowc * L + colc)[None]
            extra = jnp.any(flat_c == bi, axis=0)
            best_iou = bm_sc[0, sl]
            pos = (best_iou > IOU_THRESHOLD) | extra

            m_x0 = bm_sc[1, sl]
            m_y0 = bm_sc[2, sl]
            m_x1 = bm_sc[3, sl]
            m_y1 = bm_sc[4, sl]
            px0 = p_ref[b, 0, sl]
            py0 = p_ref[b, 1, sl]
            px1 = p_ref[b, 2, sl]
            py1 = p_ref[b, 3, sl]
            loc = (sl1((px0 + px1) / 2.0 - (m_x0 + m_x1) / 2.0)
                   + sl1((py0 + py1) / 2.0 - (m_y0 + m_y1) / 2.0)
                   + sl1((px1 - px0) - (m_x1 - m_x0))
                   + sl1((py1 - py0) - (m_y1 - m_y0)))
            loc_sum += jnp.sum(jnp.where(pos, loc, 0.0))
            n_pos += jnp.sum(jnp.where(pos, 1.0, 0.0))
            p = c_ref[b, sl]
            pos_loss += jnp.sum(jnp.where(pos, -jnp.log(p), 0.0))
            if padded:
                dead = pos | (rowc * L + colc >= n_valid)
            else:
                dead = pos
            neg_sc[gb, sl] = jnp.where(dead, 0.0, -jnp.log(1.0 - p))

        stats_sc[gb] = (jnp.where(lane == 0, n_pos, 0.0)
                        + jnp.where(lane == 1, loc_sum, 0.0)
                        + jnp.where(lane == 2, pos_loss, 0.0))[0]

    @pl.when(pl.program_id(0) == pl.num_programs(0) - 1)
    def _mine():
        neg = neg_sc[...]
        nbits = jax.lax.bitcast_convert_type(neg, jnp.int32)
        st = stats_sc[...]
        np_f = st[:, :, 0:1]
        loc_s = st[:, :, 1:2]
        pos_s = st[:, :, 2:3]
        np_i = np_f.astype(jnp.int32)
        k = jnp.minimum(np_i * NEG_POS_RATIO, n_valid - np_i)

        lo0 = jnp.zeros_like(k)
        hi0 = jnp.full_like(k, jnp.int32(0x7F000000))

        def body(_, carry):
            lo, hi = carry
            mid = lo + (hi - lo) // 2
            cnt = jnp.sum((nbits >= mid).astype(jnp.int32), axis=(1, 2),
                          keepdims=True)
            ge = cnt >= k
            return jnp.where(ge, mid, lo), jnp.where(ge, hi, mid)

        lo, _ = jax.lax.fori_loop(0, 32, body, (lo0, hi0))
        t = jax.lax.bitcast_convert_type(lo, f32)

        gtm = neg > t
        cnt_gt = jnp.sum(jnp.where(gtm, 1.0, 0.0), axis=(1, 2),
                         keepdims=True)
        sum_gt = jnp.sum(jnp.where(gtm, neg, 0.0), axis=(1, 2),
                         keepdims=True)
        kf = k.astype(f32)
        neg_sum = sum_gt + (kf - cnt_gt) * t
        neg_sum = jnp.where(k > 0, neg_sum, 0.0)

        conf_b = (pos_s / jnp.maximum(np_f, 1.0)
                  + neg_sum / jnp.maximum(kf, 1.0))
        total_conf = jnp.sum(conf_b)
        total_loc = jnp.sum(loc_s)
        total_pos = jnp.sum(np_f)
        loss = (total_loc / jnp.maximum(total_pos, 1.0)
                + total_conf / batch)
        out_ref[...] = loss + jnp.zeros((1, LANE), f32)


def kernel(bbox_pred, conf_pred, anchors, gt_boxes):
    B, N = conf_pred.shape
    G = gt_boxes.shape[1]
    L = LANE
    S = (-(-N // L) + 7) // 8 * 8
    pad = S * L - N
    MB = 4 if B % 4 == 0 else (2 if B % 2 == 0 else 1)

    a_t = anchors.transpose(0, 2, 1)
    p_t = bbox_pred.transpose(0, 2, 1)
    c = conf_pred
    if pad:
        a_t = jnp.pad(a_t, ((0, 0), (0, 0), (0, pad)), constant_values=-1.0)
        p_t = jnp.pad(p_t, ((0, 0), (0, 0), (0, pad)))
        c = jnp.pad(c, ((0, 0), (0, pad)), constant_values=0.5)
    a_t = a_t.reshape(B, 4, S, L)
    p_t = p_t.reshape(B, 4, S, L)
    c_r = c.reshape(B, S, L)

    CH = 32 if S % 32 == 0 else 8
    out = pl.pallas_call(
        functools.partial(_match_kernel, n_valid=N, n_gt=G, chunk=CH,
                          batch=B),
        grid=(B // MB,),
        in_specs=[
            pl.BlockSpec((MB, 4, S, L), lambda i: (i, 0, 0, 0)),
            pl.BlockSpec((MB, 4, S, L), lambda i: (i, 0, 0, 0)),
            pl.BlockSpec((MB, S, L), lambda i: (i, 0, 0)),
            pl.BlockSpec((MB, G, 4), lambda i: (i, 0, 0)),
        ],
        out_specs=pl.BlockSpec((1, LANE), lambda i: (0, 0)),
        out_shape=jax.ShapeDtypeStruct((1, LANE), jnp.float32),
        scratch_shapes=[pltpu.VMEM((G, S, L), jnp.float32),
                        pltpu.VMEM((5, S, L), jnp.float32),
                        pltpu.VMEM((B, S, L), jnp.float32),
                        pltpu.VMEM((B, 1, LANE), jnp.float32)],
    )(a_t, p_t, c_r, gt_boxes)
    return out[0, 0]


# speedup vs baseline: 1.0072x; 1.0072x over previous
import functools

import jax
import jax.numpy as jnp
from jax.experimental import pallas as pl
from jax.experimental.pallas import tpu as pltpu

IOU_THRESHOLD = 0.5
NEG_POS_RATIO = 3
LANE = 128


def _match_kernel(a_ref, p_ref, c_ref, gt_ref, out_ref, iou_sc, bm_sc,
                  neg_sc, stats_sc, *, n_valid, n_gt, chunk, batch):
    MB, S, L = a_ref.shape[0], a_ref.shape[2], a_ref.shape[3]
    f32 = jnp.float32
    padded = S * L > n_valid
    CH = chunk
    n_ch = S // CH

    lane = jax.lax.broadcasted_iota(jnp.int32, (1, 1, LANE), 2)

    def sl1(d):
        d = jnp.abs(d)
        return jnp.where(d < 1.0, 0.5 * d * d, d - 0.5)

    for b in range(MB):
        gb = pl.program_id(0) * MB + b
        gtv = gt_ref[b]
        gcoord = [[gtv[g:g + 1, c:c + 1] for c in range(4)]
                  for g in range(n_gt)]
        gareas = [(gc[2] - gc[0]) * (gc[3] - gc[1]) for gc in gcoord]
        for ci in range(n_ch):
            cs = ci * CH
            sl = slice(cs, cs + CH)
            ax0 = a_ref[b, 0, sl]
            ay0 = a_ref[b, 1, sl]
            ax1 = a_ref[b, 2, sl]
            ay1 = a_ref[b, 3, sl]
            area_a = (ax1 - ax0) * (ay1 - ay0)

            best_iou = jnp.full((CH, L), -1.0, f32)
            m_x0 = jnp.zeros((CH, L), f32)
            m_y0 = jnp.zeros((CH, L), f32)
            m_x1 = jnp.zeros((CH, L), f32)
            m_y1 = jnp.zeros((CH, L), f32)

            for g in range(n_gt):
                gx0, gy0, gx1, gy1 = gcoord[g]
                w = jnp.maximum(
                    jnp.minimum(ax1, gx1) - jnp.maximum(ax0, gx0), 0.0)
                h = jnp.maximum(
                    jnp.minimum(ay1, gy1) - jnp.maximum(ay0, gy0), 0.0)
                inter = w * h
                iou = inter / (area_a + gareas[g] - inter + 1e-9)
                iou_sc[g, sl] = iou
                upd = iou > best_iou
                best_iou = jnp.where(upd, iou, best_iou)
                m_x0 = jnp.where(upd, gx0, m_x0)
                m_y0 = jnp.where(upd, gy0, m_y0)
                m_x1 = jnp.where(upd, gx1, m_x1)
                m_y1 = jnp.where(upd, gy1, m_y1)

            bm_sc[0, sl] = best_iou
            bm_sc[1, sl] = m_x0
            bm_sc[2, sl] = m_y0
            bm_sc[3, sl] = m_x1
            bm_sc[4, sl] = m_y1

        row = jax.lax.broadcasted_iota(jnp.int32, (S, L), 0)
        col = jax.lax.broadcasted_iota(jnp.int32, (S, L), 1)
        fi3 = (row * L + col)[None]
        iou_all = iou_sc[...]
        gmax = jnp.max(iou_all, axis=(1, 2), keepdims=True)
        cand = jnp.where(iou_all == gmax, fi3, jnp.int32(2 ** 30))
        bi = jnp.min(cand, axis=(1, 2), keepdims=True)

        loc_sum = jnp.float32(0.0)
        n_pos = jnp.float32(0.0)
        pos_loss = jnp.float32(0.0)
        for ci in range(n_ch):
            cs = ci * CH
            sl = slice(cs, cs + CH)
            rowc = jax.lax.broadcasted_iota(jnp.int32, (CH, L), 0) + cs
            colc = jax.lax.broadcasted_iota(jnp.int32, (CH, L), 1)
            flat_c = (rowc * L + colc)[None]
            extra = jnp.any(flat_c == bi, axis=0)
            best_iou = bm_sc[0, sl]
            pos = (best_iou > IOU_THRESHOLD) | extra

            m_x0 = bm_sc[1, sl]
            m_y0 = bm_sc[2, sl]
            m_x1 = bm_sc[3, sl]
            m_y1 = bm_sc[4, sl]
            px0 = p_ref[b, 0, sl]
            py0 = p_ref[b, 1, sl]
            px1 = p_ref[b, 2, sl]
            py1 = p_ref[b, 3, sl]
            loc = (sl1((px0 + px1) / 2.0 - (m_x0 + m_x1) / 2.0)
                   + sl1((py0 + py1) / 2.0 - (m_y0 + m_y1) / 2.0)
                   + sl1((px1 - px0) - (m_x1 - m_x0))
                   + sl1((py1 - py0) - (m_y1 - m_y0)))
            loc_sum += jnp.sum(jnp.where(pos, loc, 0.0))
            n_pos += jnp.sum(jnp.where(pos, 1.0, 0.0))
            p = c_ref[b, sl]
            pos_loss += jnp.sum(jnp.where(pos, -jnp.log(p), 0.0))
            if padded:
                dead = pos | (rowc * L + colc >= n_valid)
            else:
                dead = pos
            neg_sc[gb, sl] = jnp.where(dead, 0.0, -jnp.log(1.0 - p))

        stats_sc[gb] = (jnp.where(lane == 0, n_pos, 0.0)
                        + jnp.where(lane == 1, loc_sum, 0.0)
                        + jnp.where(lane == 2, pos_loss, 0.0))[0]

    @pl.when(pl.program_id(0) == pl.num_programs(0) - 1)
    def _mine():
        neg = neg_sc[...]
        nbits = jax.lax.bitcast_convert_type(neg, jnp.int32)
        st = stats_sc[...]
        np_f = st[:, :, 0:1]
        loc_s = st[:, :, 1:2]
        pos_s = st[:, :, 2:3]
        np_i = np_f.astype(jnp.int32)
        k = jnp.minimum(np_i * NEG_POS_RATIO, n_valid - np_i)

        lo0 = jnp.zeros_like(k)
        hi0 = jnp.full_like(k, jnp.int32(0x7F000000))

        def body(_, carry):
            lo, hi = carry
            mid = lo + (hi - lo) // 2
            cnt = jnp.sum((nbits >= mid).astype(jnp.int32), axis=(1, 2),
                          keepdims=True)
            ge = cnt >= k
            return jnp.where(ge, mid, lo), jnp.where(ge, hi, mid)

        lo, _ = jax.lax.fori_loop(0, 32, body, (lo0, hi0))
        t = jax.lax.bitcast_convert_type(lo, f32)

        gtm = neg > t
        cnt_gt = jnp.sum(jnp.where(gtm, 1.0, 0.0), axis=(1, 2),
                         keepdims=True)
        sum_gt = jnp.sum(jnp.where(gtm, neg, 0.0), axis=(1, 2),
                         keepdims=True)
        kf = k.astype(f32)
        neg_sum = sum_gt + (kf - cnt_gt) * t
        neg_sum = jnp.where(k > 0, neg_sum, 0.0)

        conf_b = (pos_s / jnp.maximum(np_f, 1.0)
                  + neg_sum / jnp.maximum(kf, 1.0))
        total_conf = jnp.sum(conf_b)
        total_loc = jnp.sum(loc_s)
        total_pos = jnp.sum(np_f)
        loss = (total_loc / jnp.maximum(total_pos, 1.0)
                + total_conf / batch)
        out_ref[...] = loss + jnp.zeros((1, LANE), f32)


def kernel(bbox_pred, conf_pred, anchors, gt_boxes):
    B, N = conf_pred.shape
    G = gt_boxes.shape[1]
    L = LANE
    S = (-(-N // L) + 7) // 8 * 8
    pad = S * L - N
    MB = 2 if B % 2 == 0 else 1

    a_t = anchors.transpose(0, 2, 1)
    p_t = bbox_pred.transpose(0, 2, 1)
    c = conf_pred
    if pad:
        a_t = jnp.pad(a_t, ((0, 0), (0, 0), (0, pad)), constant_values=-1.0)
        p_t = jnp.pad(p_t, ((0, 0), (0, 0), (0, pad)))
        c = jnp.pad(c, ((0, 0), (0, pad)), constant_values=0.5)
    a_t = a_t.reshape(B, 4, S, L)
    p_t = p_t.reshape(B, 4, S, L)
    c_r = c.reshape(B, S, L)

    CH = 32 if S % 32 == 0 else 8
    out = pl.pallas_call(
        functools.partial(_match_kernel, n_valid=N, n_gt=G, chunk=CH,
                          batch=B),
        grid=(B // MB,),
        in_specs=[
            pl.BlockSpec((MB, 4, S, L), lambda i: (i, 0, 0, 0)),
            pl.BlockSpec((MB, 4, S, L), lambda i: (i, 0, 0, 0)),
            pl.BlockSpec((MB, S, L), lambda i: (i, 0, 0)),
            pl.BlockSpec((MB, G, 4), lambda i: (i, 0, 0)),
        ],
        out_specs=pl.BlockSpec((1, LANE), lambda i: (0, 0)),
        out_shape=jax.ShapeDtypeStruct((1, LANE), jnp.float32),
        scratch_shapes=[pltpu.VMEM((G, S, L), jnp.float32),
                        pltpu.VMEM((5, S, L), jnp.float32),
                        pltpu.VMEM((B, S, L), jnp.float32),
                        pltpu.VMEM((B, 1, LANE), jnp.float32)],
    )(a_t, p_t, c_r, gt_boxes)
    return out[0, 0]


# speedup vs baseline: 1.0177x; 1.0104x over previous
import functools

import jax
import jax.numpy as jnp
from jax.experimental import pallas as pl
from jax.experimental.pallas import tpu as pltpu

IOU_THRESHOLD = 0.5
NEG_POS_RATIO = 3
LANE = 128


def _match_kernel(a_ref, p_ref, c_ref, gt_ref, out_ref, iou_sc, bm_sc,
                  neg_sc, stats_sc, *, n_valid, n_gt, chunk, batch):
    MB, S, L = a_ref.shape[0], a_ref.shape[2], a_ref.shape[3]
    f32 = jnp.float32
    padded = S * L > n_valid
    CH = chunk
    n_ch = S // CH

    lane = jax.lax.broadcasted_iota(jnp.int32, (1, 1, LANE), 2)

    def sl1(d):
        d = jnp.abs(d)
        return jnp.where(d < 1.0, 0.5 * d * d, d - 0.5)

    for b in range(MB):
        gb = pl.program_id(0) * MB + b
        gtv = gt_ref[b]
        gcoord = [[gtv[g:g + 1, c:c + 1] for c in range(4)]
                  for g in range(n_gt)]
        gareas = [(gc[2] - gc[0]) * (gc[3] - gc[1]) for gc in gcoord]
        for ci in range(n_ch):
            cs = ci * CH
            sl = slice(cs, cs + CH)
            ax0 = a_ref[b, 0, sl]
            ay0 = a_ref[b, 1, sl]
            ax1 = a_ref[b, 2, sl]
            ay1 = a_ref[b, 3, sl]
            area_a = (ax1 - ax0) * (ay1 - ay0)

            best_iou = jnp.full((CH, L), -1.0, f32)
            m_x0 = jnp.zeros((CH, L), f32)
            m_y0 = jnp.zeros((CH, L), f32)
            m_x1 = jnp.zeros((CH, L), f32)
            m_y1 = jnp.zeros((CH, L), f32)

            for g in range(n_gt):
                gx0, gy0, gx1, gy1 = gcoord[g]
                w = jnp.maximum(
                    jnp.minimum(ax1, gx1) - jnp.maximum(ax0, gx0), 0.0)
                h = jnp.maximum(
                    jnp.minimum(ay1, gy1) - jnp.maximum(ay0, gy0), 0.0)
                inter = w * h
                iou = inter / (area_a + gareas[g] - inter + 1e-9)
                iou_sc[g, sl] = iou
                upd = iou > best_iou
                best_iou = jnp.where(upd, iou, best_iou)
                m_x0 = jnp.where(upd, gx0, m_x0)
                m_y0 = jnp.where(upd, gy0, m_y0)
                m_x1 = jnp.where(upd, gx1, m_x1)
                m_y1 = jnp.where(upd, gy1, m_y1)

            bm_sc[0, sl] = best_iou
            bm_sc[1, sl] = m_x0
            bm_sc[2, sl] = m_y0
            bm_sc[3, sl] = m_x1
            bm_sc[4, sl] = m_y1

        row = jax.lax.broadcasted_iota(jnp.int32, (S, L), 0)
        col = jax.lax.broadcasted_iota(jnp.int32, (S, L), 1)
        fi3 = (row * L + col)[None]
        iou_all = iou_sc[...]
        gmax = jnp.max(iou_all, axis=(1, 2), keepdims=True)
        cand = jnp.where(iou_all == gmax, fi3, jnp.int32(2 ** 30))
        bi = jnp.min(cand, axis=(1, 2), keepdims=True)

        loc_sum = jnp.float32(0.0)
        n_pos = jnp.float32(0.0)
        pos_loss = jnp.float32(0.0)
        for ci in range(n_ch):
            cs = ci * CH
            sl = slice(cs, cs + CH)
            rowc = jax.lax.broadcasted_iota(jnp.int32, (CH, L), 0) + cs
            colc = jax.lax.broadcasted_iota(jnp.int32, (CH, L), 1)
            flat_c = (rowc * L + colc)[None]
            extra = jnp.any(flat_c == bi, axis=0)
            best_iou = bm_sc[0, sl]
            pos = (best_iou > IOU_THRESHOLD) | extra

            m_x0 = bm_sc[1, sl]
            m_y0 = bm_sc[2, sl]
            m_x1 = bm_sc[3, sl]
            m_y1 = bm_sc[4, sl]
            px0 = p_ref[b, 0, sl]
            py0 = p_ref[b, 1, sl]
            px1 = p_ref[b, 2, sl]
            py1 = p_ref[b, 3, sl]
            loc = (sl1((px0 + px1) / 2.0 - (m_x0 + m_x1) / 2.0)
                   + sl1((py0 + py1) / 2.0 - (m_y0 + m_y1) / 2.0)
                   + sl1((px1 - px0) - (m_x1 - m_x0))
                   + sl1((py1 - py0) - (m_y1 - m_y0)))
            loc_sum += jnp.sum(jnp.where(pos, loc, 0.0))
            n_pos += jnp.sum(jnp.where(pos, 1.0, 0.0))
            p = c_ref[b, sl]
            pos_loss += jnp.sum(jnp.where(pos, -jnp.log(p), 0.0))
            if padded:
                dead = pos | (rowc * L + colc >= n_valid)
            else:
                dead = pos
            neg_sc[gb, sl] = jnp.where(dead, 0.0, -jnp.log(1.0 - p))

        stats_sc[gb] = (jnp.where(lane == 0, n_pos, 0.0)
                        + jnp.where(lane == 1, loc_sum, 0.0)
                        + jnp.where(lane == 2, pos_loss, 0.0))[0]

    @pl.when(pl.program_id(0) == pl.num_programs(0) - 1)
    def _mine():
        neg = neg_sc[...]
        nbits = jax.lax.bitcast_convert_type(neg, jnp.int32)
        st = stats_sc[...]
        np_f = st[:, :, 0:1]
        loc_s = st[:, :, 1:2]
        pos_s = st[:, :, 2:3]
        np_i = np_f.astype(jnp.int32)
        k = jnp.minimum(np_i * NEG_POS_RATIO, n_valid - np_i)

        lo0 = jnp.zeros_like(k)
        hi0 = jnp.full_like(k, jnp.int32(0x7F000000))

        def body(_, carry):
            lo, hi = carry
            mid = lo + (hi - lo) // 2
            cnt = jnp.sum((nbits >= mid).astype(jnp.int32), axis=(1, 2),
                          keepdims=True)
            ge = cnt >= k
            return jnp.where(ge, mid, lo), jnp.where(ge, hi, mid)

        lo, _ = jax.lax.fori_loop(0, 32, body, (lo0, hi0))
        t = jax.lax.bitcast_convert_type(lo, f32)

        gtm = neg > t
        cnt_gt = jnp.sum(jnp.where(gtm, 1.0, 0.0), axis=(1, 2),
                         keepdims=True)
        sum_gt = jnp.sum(jnp.where(gtm, neg, 0.0), axis=(1, 2),
                         keepdims=True)
        kf = k.astype(f32)
        neg_sum = sum_gt + (kf - cnt_gt) * t
        neg_sum = jnp.where(k > 0, neg_sum, 0.0)

        conf_b = (pos_s / jnp.maximum(np_f, 1.0)
                  + neg_sum / jnp.maximum(kf, 1.0))
        total_conf = jnp.sum(conf_b)
        total_loc = jnp.sum(loc_s)
        total_pos = jnp.sum(np_f)
        loss = (total_loc / jnp.maximum(total_pos, 1.0)
                + total_conf / batch)
        out_ref[...] = loss + jnp.zeros((1, LANE), f32)


def kernel(bbox_pred, conf_pred, anchors, gt_boxes):
    B, N = conf_pred.shape
    G = gt_boxes.shape[1]
    L = LANE
    S = (-(-N // L) + 7) // 8 * 8
    pad = S * L - N
    MB = 2 if B % 2 == 0 else 1

    a, p, c = anchors, bbox_pred, conf_pred
    if pad:
        a = jnp.pad(a, ((0, 0), (0, pad), (0, 0)), constant_values=-1.0)
        p = jnp.pad(p, ((0, 0), (0, pad), (0, 0)))
        c = jnp.pad(c, ((0, 0), (0, pad)), constant_values=0.5)
    a_t = a.transpose(0, 2, 1).reshape(B, 4, S, L)
    p_t = p.transpose(0, 2, 1).reshape(B, 4, S, L)
    c_r = c.reshape(B, S, L)

    CH = 32 if S % 32 == 0 else 8
    out = pl.pallas_call(
        functools.partial(_match_kernel, n_valid=N, n_gt=G, chunk=CH,
                          batch=B),
        grid=(B // MB,),
        in_specs=[
            pl.BlockSpec((MB, 4, S, L), lambda i: (i, 0, 0, 0)),
            pl.BlockSpec((MB, 4, S, L), lambda i: (i, 0, 0, 0)),
            pl.BlockSpec((MB, S, L), lambda i: (i, 0, 0)),
            pl.BlockSpec((MB, G, 4), lambda i: (i, 0, 0)),
        ],
        out_specs=pl.BlockSpec((1, LANE), lambda i: (0, 0)),
        out_shape=jax.ShapeDtypeStruct((1, LANE), jnp.float32),
        scratch_shapes=[pltpu.VMEM((G, S, L), jnp.float32),
                        pltpu.VMEM((5, S, L), jnp.float32),
                        pltpu.VMEM((B, S, L), jnp.float32),
                        pltpu.VMEM((B, 1, LANE), jnp.float32)],
    )(a_t, p_t, c_r, gt_boxes)
    return out[0, 0]
